# Initial kernel scaffold; baseline (speedup 1.0000x reference)
import functools
import math

import jax
import jax.numpy as jnp
from jax import lax
from jax.experimental import pallas as pl
from jax.experimental.pallas import tpu as pltpu
from jax.experimental.pallas import tpu_sc as plsc

NC = 2
NS = 16
NW = NC * NS
L = 16

_EPS = 1e-5
_NEG = -1e30


@functools.lru_cache(maxsize=None)
def _make_deg_kernel(E, N):
    EPW = E // NW
    assert E % NW == 0 and EPW % L == 0 and N % L == 0
    mesh = plsc.VectorSubcoreMesh(core_axis_name="c", subcore_axis_name="s")

    @functools.partial(
        pl.kernel,
        out_type=jax.ShapeDtypeStruct((NW, N), jnp.float32),
        mesh=mesh,
        scratch_types=[
            pltpu.VMEM((EPW,), jnp.int32),
            pltpu.VMEM((N,), jnp.float32),
        ],
    )
    def deg_kernel(dst_hbm, out_hbm, idx_v, hist_v):
        c = lax.axis_index("c")
        s = lax.axis_index("s")
        wid = s * NC + c
        zeros = jnp.zeros((L,), jnp.float32)

        def zero_body(i, carry):
            hist_v[pl.ds(i * L, L)] = zeros
            return carry

        lax.fori_loop(0, N // L, zero_body, 0)
        pltpu.sync_copy(dst_hbm.at[pl.ds(wid * EPW, EPW)], idx_v)
        ones = jnp.ones((L,), jnp.float32)

        def body(i, carry):
            idx = idx_v[pl.ds(i * L, L)]
            plsc.addupdate_scatter(hist_v, [idx], ones)
            return carry

        lax.fori_loop(0, EPW // L, body, 0)
        pltpu.sync_copy(hist_v, out_hbm.at[wid])

    return deg_kernel


@functools.lru_cache(maxsize=None)
def _make_scatter_kernel(E, N, D, K=80, ZR=25):
    EPW = E // NW
    RPT = N // NS
    assert E % NW == 0 and EPW % K == 0 and K % 8 == 0 and K <= 128
    assert N % NS == 0 and RPT % ZR == 0 and D % L == 0
    mesh = plsc.VectorSubcoreMesh(core_axis_name="c", subcore_axis_name="s")

    @functools.partial(
        pl.kernel,
        out_type=jax.ShapeDtypeStruct((NC, N, D), jnp.float32),
        mesh=mesh,
        scratch_types=[
            pltpu.VMEM((K,), jnp.int32),
            pltpu.VMEM((K,), jnp.int32),
            pltpu.VMEM((K, D), jnp.float32),
            pltpu.VMEM((ZR, D), jnp.float32),
            pltpu.VMEM_SHARED((N, D), jnp.float32),
            pltpu.SemaphoreType.DMA,
        ],
    )
    def scatter_kernel(t_hbm, src_hbm, dst_hbm, out_hbm,
                       src_v, dst_v, rows_v, zero_v, acc_sh, sem):
        c = lax.axis_index("c")
        s = lax.axis_index("s")
        wid = s * NC + c
        zeros = jnp.zeros((L,), jnp.float32)
        for r in range(ZR):
            for j in range(D // L):
                zero_v[r, pl.ds(j * L, L)] = zeros

        def zero_body(i, carry):
            pltpu.sync_copy(zero_v, acc_sh.at[pl.ds(s * RPT + i * ZR, ZR)])
            return carry

        lax.fori_loop(0, RPT // ZR, zero_body, 0)
        plsc.subcore_barrier()

        base = wid * EPW

        def body(i, carry):
            off = base + i * K
            pltpu.sync_copy(src_hbm.at[pl.ds(off, K)], src_v)
            pltpu.sync_copy(dst_hbm.at[pl.ds(off, K)], dst_v)
            pltpu.async_copy(t_hbm.at[src_v], rows_v, sem).wait()
            pltpu.sync_copy(rows_v, acc_sh.at[dst_v], add=True)
            return carry

        lax.fori_loop(0, EPW // K, body, 0)
        plsc.subcore_barrier()
        pltpu.sync_copy(acc_sh.at[pl.ds(s * RPT, RPT)],
                        out_hbm.at[c, pl.ds(s * RPT, RPT)])

    return scatter_kernel


def _tc_prep(degp_t, x, W1, R=1000):
    N, F = x.shape
    H = W1.shape[1]
    assert N % R == 0

    def body(degp_ref, x_ref, w_ref, dinv_ref, t_ref):
        deg = jnp.sum(degp_ref[...], axis=1, keepdims=True) + 1.0
        dinv = lax.rsqrt(deg)
        dinv_ref[...] = dinv
        t_ref[...] = jnp.dot(x_ref[...], w_ref[...],
                             preferred_element_type=jnp.float32) * dinv

    return pl.pallas_call(
        body,
        grid=(N // R,),
        in_specs=[
            pl.BlockSpec((R, NW), lambda i: (i, 0)),
            pl.BlockSpec((R, F), lambda i: (i, 0)),
            pl.BlockSpec((F, H), lambda i: (0, 0)),
        ],
        out_specs=[
            pl.BlockSpec((R, 1), lambda i: (i, 0)),
            pl.BlockSpec((R, H), lambda i: (i, 0)),
        ],
        out_shape=[
            jax.ShapeDtypeStruct((N, 1), jnp.float32),
            jax.ShapeDtypeStruct((N, H), jnp.float32),
        ],
    )(degp_t, x, W1)


def _tc_mid(sp, t, dinv, b, g, be, res, W, R=1000):
    N, Hp = t.shape
    Hn = W.shape[1]
    assert N % R == 0
    bnscale = 1.0 / math.sqrt(1.0 + _EPS)
    with_res = res is not None

    def body(sp_ref, t_ref, dinv_ref, b_ref, g_ref, be_ref, *rest):
        if with_res:
            res_ref, w_ref, out_ref = rest
        else:
            w_ref, out_ref = rest
        a = sp_ref[...]
        dinv = dinv_ref[...]
        u = (a[0] + a[1] + t_ref[...]) * dinv + b_ref[...]
        u = u * (g_ref[...] * bnscale) + be_ref[...]
        if with_res:
            u = u + res_ref[...]
        u = jnp.maximum(u, 0.0)
        out_ref[...] = jnp.dot(u, w_ref[...],
                               preferred_element_type=jnp.float32) * dinv

    in_specs = [
        pl.BlockSpec((NC, R, Hp), lambda i: (0, i, 0)),
        pl.BlockSpec((R, Hp), lambda i: (i, 0)),
        pl.BlockSpec((R, 1), lambda i: (i, 0)),
        pl.BlockSpec((1, Hp), lambda i: (0, 0)),
        pl.BlockSpec((1, Hp), lambda i: (0, 0)),
        pl.BlockSpec((1, Hp), lambda i: (0, 0)),
    ]
    args = [sp, t, dinv, b, g, be]
    if with_res:
        in_specs.append(pl.BlockSpec((R, Hp), lambda i: (i, 0)))
        args.append(res)
    in_specs.append(pl.BlockSpec((Hp, Hn), lambda i: (0, 0)))
    args.append(W)

    return pl.pallas_call(
        body,
        grid=(N // R,),
        in_specs=in_specs,
        out_specs=pl.BlockSpec((R, Hn), lambda i: (i, 0)),
        out_shape=jax.ShapeDtypeStruct((N, Hn), jnp.float32),
    )(*args)


def _tc_final(sp, t, dinv, b, R=1000):
    N, Cp = t.shape
    assert N % R == 0

    def body(sp_ref, t_ref, dinv_ref, b_ref, out_ref):
        a = sp_ref[...]
        u = (a[0] + a[1] + t_ref[...]) * dinv_ref[...] + b_ref[...]
        m = jnp.max(u, axis=1, keepdims=True)
        z = u - m
        out_ref[...] = z - jnp.log(jnp.sum(jnp.exp(z), axis=1, keepdims=True))

    return pl.pallas_call(
        body,
        grid=(N // R,),
        in_specs=[
            pl.BlockSpec((NC, R, Cp), lambda i: (0, i, 0)),
            pl.BlockSpec((R, Cp), lambda i: (i, 0)),
            pl.BlockSpec((R, 1), lambda i: (i, 0)),
            pl.BlockSpec((1, Cp), lambda i: (0, 0)),
        ],
        out_specs=pl.BlockSpec((R, Cp), lambda i: (i, 0)),
        out_shape=jax.ShapeDtypeStruct((N, Cp), jnp.float32),
    )(sp, t, dinv, b)


def kernel(x, edge_index, W1, b1, g1, be1, W2, b2, g2, be2, W3, b3):
    N, F = x.shape
    H = W1.shape[1]
    C = W3.shape[1]
    E = edge_index.shape[1]
    src = edge_index[0]
    dst = edge_index[1]

    Cp = ((C + L - 1) // L) * L
    W3p = jnp.pad(W3, ((0, 0), (0, Cp - C)))
    b3p = jnp.concatenate([b3, jnp.full((Cp - C,), _NEG, jnp.float32)])

    b1r, g1r, be1r = b1[None, :], g1[None, :], be1[None, :]
    b2r, g2r, be2r = b2[None, :], g2[None, :], be2[None, :]
    b3r = b3p[None, :]

    degp = _make_deg_kernel(E, N)(dst)
    dinv, t1 = _tc_prep(degp.T, x, W1)

    scat_h = _make_scatter_kernel(E, N, H)
    scat_c = _make_scatter_kernel(E, N, Cp)

    s1 = scat_h(t1, src, dst)
    t2 = _tc_mid(s1, t1, dinv, b1r, g1r, be1r, x, W2)
    s2 = scat_h(t2, src, dst)
    t3 = _tc_mid(s2, t2, dinv, b2r, g2r, be2r, None, W3p)
    s3 = scat_c(t3, src, dst)
    out = _tc_final(s3, t3, dinv, b3r)
    return out[:, :C]


# speedup vs baseline: 11.6545x; 11.6545x over previous
import functools
import math

import jax
import jax.numpy as jnp
from jax import lax
from jax.experimental import pallas as pl
from jax.experimental.pallas import tpu as pltpu
from jax.experimental.pallas import tpu_sc as plsc

NC = 2
NS = 16
NW = NC * NS
L = 16

_EPS = 1e-5
_NEG = -1e30


@functools.lru_cache(maxsize=None)
def _make_deg_kernel(E, N):
    EPW = E // NW
    assert E % NW == 0 and EPW % L == 0 and N % L == 0
    mesh = plsc.VectorSubcoreMesh(core_axis_name="c", subcore_axis_name="s")

    @functools.partial(
        pl.kernel,
        out_type=jax.ShapeDtypeStruct((NW, N), jnp.float32),
        mesh=mesh,
        scratch_types=[
            pltpu.VMEM((EPW,), jnp.int32),
            pltpu.VMEM((N,), jnp.float32),
        ],
        compiler_params=pltpu.CompilerParams(needs_layout_passes=False),
    )
    def deg_kernel(dst_hbm, out_hbm, idx_v, hist_v):
        c = lax.axis_index("c")
        s = lax.axis_index("s")
        wid = s * NC + c
        zeros = jnp.zeros((L,), jnp.float32)

        def zero_body(i, carry):
            hist_v[pl.ds(i * L, L)] = zeros
            return carry

        lax.fori_loop(0, N // L, zero_body, 0)
        pltpu.sync_copy(dst_hbm.at[pl.ds(wid * EPW, EPW)], idx_v)
        ones = jnp.ones((L,), jnp.float32)

        def body(i, carry):
            idx = idx_v[pl.ds(i * L, L)]
            plsc.addupdate_scatter(hist_v, [idx], ones)
            return carry

        lax.fori_loop(0, EPW // L, body, 0)
        pltpu.sync_copy(hist_v, out_hbm.at[wid])

    return deg_kernel


@functools.lru_cache(maxsize=None)
def _make_scatter_kernel(E, N, D, K=80, ZR=25):
    EPW = E // NW
    RPT = N // NS
    assert E % NW == 0 and EPW % K == 0 and K % 8 == 0 and K <= 128
    assert N % NS == 0 and RPT % ZR == 0 and D % L == 0
    mesh = plsc.VectorSubcoreMesh(core_axis_name="c", subcore_axis_name="s")

    @functools.partial(
        pl.kernel,
        out_type=jax.ShapeDtypeStruct((NC, N, D), jnp.float32),
        mesh=mesh,
        scratch_types=[
            pltpu.VMEM((K,), jnp.int32),
            pltpu.VMEM((K,), jnp.int32),
            pltpu.VMEM((K, D), jnp.float32),
            pltpu.VMEM((ZR, D), jnp.float32),
            pltpu.VMEM_SHARED((N, D), jnp.float32),
            pltpu.SemaphoreType.DMA,
        ],
        compiler_params=pltpu.CompilerParams(use_tc_tiling_on_sc=False),
    )
    def scatter_kernel(t_hbm, src_hbm, dst_hbm, out_hbm,
                       src_v, dst_v, rows_v, zero_v, acc_sh, sem):
        c = lax.axis_index("c")
        s = lax.axis_index("s")
        wid = s * NC + c
        zeros = jnp.zeros((L,), jnp.float32)
        for r in range(ZR):
            for j in range(D // L):
                zero_v[r, pl.ds(j * L, L)] = zeros

        def zero_body(i, carry):
            pltpu.sync_copy(zero_v, acc_sh.at[pl.ds(s * RPT + i * ZR, ZR)])
            return carry

        lax.fori_loop(0, RPT // ZR, zero_body, 0)
        plsc.subcore_barrier()

        base = wid * EPW

        def body(i, carry):
            off = base + i * K
            pltpu.sync_copy(src_hbm.at[pl.ds(off, K)], src_v)
            pltpu.sync_copy(dst_hbm.at[pl.ds(off, K)], dst_v)
            pltpu.async_copy(t_hbm.at[src_v], rows_v, sem).wait()
            pltpu.sync_copy(rows_v, acc_sh.at[dst_v], add=True)
            return carry

        lax.fori_loop(0, EPW // K, body, 0)
        plsc.subcore_barrier()
        pltpu.sync_copy(acc_sh.at[pl.ds(s * RPT, RPT)],
                        out_hbm.at[c, pl.ds(s * RPT, RPT)])

    return scatter_kernel


def _tc_prep(degp_t, x, W1, R=1000):
    N, F = x.shape
    H = W1.shape[1]
    assert N % R == 0

    def body(degp_ref, x_ref, w_ref, dinv_ref, t_ref):
        deg = jnp.sum(degp_ref[...], axis=1, keepdims=True) + 1.0
        dinv = lax.rsqrt(deg)
        dinv_ref[...] = dinv
        t_ref[...] = jnp.dot(x_ref[...], w_ref[...],
                             preferred_element_type=jnp.float32) * dinv

    return pl.pallas_call(
        body,
        grid=(N // R,),
        in_specs=[
            pl.BlockSpec((R, NW), lambda i: (i, 0)),
            pl.BlockSpec((R, F), lambda i: (i, 0)),
            pl.BlockSpec((F, H), lambda i: (0, 0)),
        ],
        out_specs=[
            pl.BlockSpec((R, 1), lambda i: (i, 0)),
            pl.BlockSpec((R, H), lambda i: (i, 0)),
        ],
        out_shape=[
            jax.ShapeDtypeStruct((N, 1), jnp.float32),
            jax.ShapeDtypeStruct((N, H), jnp.float32),
        ],
    )(degp_t, x, W1)


def _tc_mid(sp, t, dinv, b, g, be, res, W, R=1000):
    N, Hp = t.shape
    Hn = W.shape[1]
    assert N % R == 0
    bnscale = 1.0 / math.sqrt(1.0 + _EPS)
    with_res = res is not None

    def body(sp_ref, t_ref, dinv_ref, b_ref, g_ref, be_ref, *rest):
        if with_res:
            res_ref, w_ref, out_ref = rest
        else:
            w_ref, out_ref = rest
        a = sp_ref[...]
        dinv = dinv_ref[...]
        u = (a[0] + a[1] + t_ref[...]) * dinv + b_ref[...]
        u = u * (g_ref[...] * bnscale) + be_ref[...]
        if with_res:
            u = u + res_ref[...]
        u = jnp.maximum(u, 0.0)
        out_ref[...] = jnp.dot(u, w_ref[...],
                               preferred_element_type=jnp.float32) * dinv

    in_specs = [
        pl.BlockSpec((NC, R, Hp), lambda i: (0, i, 0)),
        pl.BlockSpec((R, Hp), lambda i: (i, 0)),
        pl.BlockSpec((R, 1), lambda i: (i, 0)),
        pl.BlockSpec((1, Hp), lambda i: (0, 0)),
        pl.BlockSpec((1, Hp), lambda i: (0, 0)),
        pl.BlockSpec((1, Hp), lambda i: (0, 0)),
    ]
    args = [sp, t, dinv, b, g, be]
    if with_res:
        in_specs.append(pl.BlockSpec((R, Hp), lambda i: (i, 0)))
        args.append(res)
    in_specs.append(pl.BlockSpec((Hp, Hn), lambda i: (0, 0)))
    args.append(W)

    return pl.pallas_call(
        body,
        grid=(N // R,),
        in_specs=in_specs,
        out_specs=pl.BlockSpec((R, Hn), lambda i: (i, 0)),
        out_shape=jax.ShapeDtypeStruct((N, Hn), jnp.float32),
    )(*args)


def _tc_final(sp, t, dinv, b, R=1000):
    N, Cp = t.shape
    assert N % R == 0

    def body(sp_ref, t_ref, dinv_ref, b_ref, out_ref):
        a = sp_ref[...]
        u = (a[0] + a[1] + t_ref[...]) * dinv_ref[...] + b_ref[...]
        m = jnp.max(u, axis=1, keepdims=True)
        z = u - m
        out_ref[...] = z - jnp.log(jnp.sum(jnp.exp(z), axis=1, keepdims=True))

    return pl.pallas_call(
        body,
        grid=(N // R,),
        in_specs=[
            pl.BlockSpec((NC, R, Cp), lambda i: (0, i, 0)),
            pl.BlockSpec((R, Cp), lambda i: (i, 0)),
            pl.BlockSpec((R, 1), lambda i: (i, 0)),
            pl.BlockSpec((1, Cp), lambda i: (0, 0)),
        ],
        out_specs=pl.BlockSpec((R, Cp), lambda i: (i, 0)),
        out_shape=jax.ShapeDtypeStruct((N, Cp), jnp.float32),
    )(sp, t, dinv, b)


def kernel(x, edge_index, W1, b1, g1, be1, W2, b2, g2, be2, W3, b3):
    N, F = x.shape
    H = W1.shape[1]
    C = W3.shape[1]
    E = edge_index.shape[1]
    src = edge_index[0]
    dst = edge_index[1]

    Cp = ((C + L - 1) // L) * L
    W3p = jnp.pad(W3, ((0, 0), (0, Cp - C)))
    b3p = jnp.concatenate([b3, jnp.full((Cp - C,), _NEG, jnp.float32)])

    b1r, g1r, be1r = b1[None, :], g1[None, :], be1[None, :]
    b2r, g2r, be2r = b2[None, :], g2[None, :], be2[None, :]
    b3r = b3p[None, :]

    degp = _make_deg_kernel(E, N)(dst)
    dinv, t1 = _tc_prep(degp.T, x, W1)

    scat_h = _make_scatter_kernel(E, N, H)
    scat_c = _make_scatter_kernel(E, N, Cp)

    s1 = scat_h(t1, src, dst)
    t2 = _tc_mid(s1, t1, dinv, b1r, g1r, be1r, x, W2)
    s2 = scat_h(t2, src, dst)
    t3 = _tc_mid(s2, t2, dinv, b2r, g2r, be2r, None, W3p)
    s3 = scat_c(t3, src, dst)
    out = _tc_final(s3, t3, dinv, b3r)
    return out[:, :C]


# speedup vs baseline: 18.9648x; 1.6273x over previous
import functools
import math

import jax
import jax.numpy as jnp
from jax import lax
from jax.experimental import pallas as pl
from jax.experimental.pallas import tpu as pltpu
from jax.experimental.pallas import tpu_sc as plsc

NC = 2
NS = 16
NW = NC * NS
L = 16

_EPS = 1e-5
_NEG = -1e30


@functools.lru_cache(maxsize=None)
def _make_deg_kernel(E, N):
    EPW = E // NW
    assert E % NW == 0 and EPW % L == 0 and N % L == 0
    mesh = plsc.VectorSubcoreMesh(core_axis_name="c", subcore_axis_name="s")

    @functools.partial(
        pl.kernel,
        out_type=jax.ShapeDtypeStruct((NW, N), jnp.float32),
        mesh=mesh,
        scratch_types=[
            pltpu.VMEM((EPW,), jnp.int32),
            pltpu.VMEM((N,), jnp.float32),
        ],
        compiler_params=pltpu.CompilerParams(needs_layout_passes=False),
    )
    def deg_kernel(dst_hbm, out_hbm, idx_v, hist_v):
        c = lax.axis_index("c")
        s = lax.axis_index("s")
        wid = s * NC + c
        zeros = jnp.zeros((L,), jnp.float32)

        def zero_body(i, carry):
            hist_v[pl.ds(i * L, L)] = zeros
            return carry

        lax.fori_loop(0, N // L, zero_body, 0)
        pltpu.sync_copy(dst_hbm.at[pl.ds(wid * EPW, EPW)], idx_v)
        ones = jnp.ones((L,), jnp.float32)

        def body(i, carry):
            idx = idx_v[pl.ds(i * L, L)]
            plsc.addupdate_scatter(hist_v, [idx], ones)
            return carry

        lax.fori_loop(0, EPW // L, body, 0)
        pltpu.sync_copy(hist_v, out_hbm.at[wid])

    return deg_kernel


@functools.lru_cache(maxsize=None)
def _make_scatter_kernel(E, N, D, K=40, ZR=25, NB=2):
    EPW = E // NW
    NCH = EPW // K
    RPT = N // NS
    assert E % NW == 0 and EPW % K == 0 and K % 8 == 0 and K <= 128
    assert N % NS == 0 and RPT % ZR == 0 and D % L == 0 and NCH % NB == 0
    assert K >= ZR
    mesh = plsc.VectorSubcoreMesh(core_axis_name="c", subcore_axis_name="s")

    @functools.partial(
        pl.kernel,
        out_type=jax.ShapeDtypeStruct((NC, N, D), jnp.float32),
        mesh=mesh,
        scratch_types=[
            pltpu.VMEM((NCH, K), jnp.int32),
            pltpu.VMEM((NCH, K), jnp.int32),
            [pltpu.VMEM((K, D), jnp.float32)] * NB,
            pltpu.VMEM_SHARED((N, D), jnp.float32),
            [pltpu.SemaphoreType.DMA] * NB,
            [pltpu.SemaphoreType.DMA] * NB,
        ],
        compiler_params=pltpu.CompilerParams(use_tc_tiling_on_sc=False),
    )
    def scatter_kernel(t_hbm, src_hbm, dst_hbm, out_hbm,
                       srcall, dstall, rows, acc_sh, gsem, ssem):
        c = lax.axis_index("c")
        s = lax.axis_index("s")
        wid = s * NC + c
        pltpu.sync_copy(src_hbm.at[pl.ds(wid * NCH, NCH)], srcall)
        pltpu.sync_copy(dst_hbm.at[pl.ds(wid * NCH, NCH)], dstall)
        zeros = jnp.zeros((L,), jnp.float32)

        def zfill_body(r, carry):
            for j in range(D // L):
                rows[0][r, pl.ds(j * L, L)] = zeros
            return carry

        lax.fori_loop(0, ZR, zfill_body, 0)

        def zero_body(i, carry):
            pltpu.sync_copy(rows[0].at[pl.ds(0, ZR)],
                            acc_sh.at[pl.ds(s * RPT + i * ZR, ZR)])
            return carry

        lax.fori_loop(0, RPT // ZR, zero_body, 0)
        plsc.subcore_barrier()

        for b in range(NB):
            pltpu.async_copy(t_hbm.at[srcall.at[b]], rows[b], gsem[b])

        NP = NCH // NB

        def body(p, carry):
            c0 = p * NB
            for b in range(NB):
                pltpu.make_async_copy(
                    t_hbm.at[srcall.at[c0 + b]], rows[b], gsem[b]).wait()
                pltpu.async_copy(
                    rows[b], acc_sh.at[dstall.at[c0 + b]], ssem[b], add=True)

            @pl.when(p < NP - 1)
            def _prefetch():
                for b in range(NB):
                    pltpu.make_async_copy(
                        rows[b], acc_sh.at[dstall.at[c0 + b]], ssem[b]).wait()
                    pltpu.async_copy(
                        t_hbm.at[srcall.at[c0 + NB + b]], rows[b], gsem[b])

            return carry

        lax.fori_loop(0, NP, body, 0)
        for b in range(NB):
            pltpu.make_async_copy(
                rows[b], acc_sh.at[dstall.at[NCH - NB + b]], ssem[b]).wait()
        plsc.subcore_barrier()
        pltpu.sync_copy(acc_sh.at[pl.ds(s * RPT, RPT)],
                        out_hbm.at[c, pl.ds(s * RPT, RPT)])

    return scatter_kernel


def _tc_prep(degp_t, x, W1, R=1000):
    N, F = x.shape
    H = W1.shape[1]
    assert N % R == 0

    def body(degp_ref, x_ref, w_ref, dinv_ref, t_ref):
        deg = jnp.sum(degp_ref[...], axis=1, keepdims=True) + 1.0
        dinv = lax.rsqrt(deg)
        dinv_ref[...] = dinv
        t_ref[...] = jnp.dot(x_ref[...], w_ref[...],
                             preferred_element_type=jnp.float32) * dinv

    return pl.pallas_call(
        body,
        grid=(N // R,),
        in_specs=[
            pl.BlockSpec((R, NW), lambda i: (i, 0)),
            pl.BlockSpec((R, F), lambda i: (i, 0)),
            pl.BlockSpec((F, H), lambda i: (0, 0)),
        ],
        out_specs=[
            pl.BlockSpec((R, 1), lambda i: (i, 0)),
            pl.BlockSpec((R, H), lambda i: (i, 0)),
        ],
        out_shape=[
            jax.ShapeDtypeStruct((N, 1), jnp.float32),
            jax.ShapeDtypeStruct((N, H), jnp.float32),
        ],
    )(degp_t, x, W1)


def _tc_mid(sp, t, dinv, b, g, be, res, W, R=1000):
    N, Hp = t.shape
    Hn = W.shape[1]
    assert N % R == 0
    bnscale = 1.0 / math.sqrt(1.0 + _EPS)
    with_res = res is not None

    def body(sp_ref, t_ref, dinv_ref, b_ref, g_ref, be_ref, *rest):
        if with_res:
            res_ref, w_ref, out_ref = rest
        else:
            w_ref, out_ref = rest
        a = sp_ref[...]
        dinv = dinv_ref[...]
        u = (a[0] + a[1] + t_ref[...]) * dinv + b_ref[...]
        u = u * (g_ref[...] * bnscale) + be_ref[...]
        if with_res:
            u = u + res_ref[...]
        u = jnp.maximum(u, 0.0)
        out_ref[...] = jnp.dot(u, w_ref[...],
                               preferred_element_type=jnp.float32) * dinv

    in_specs = [
        pl.BlockSpec((NC, R, Hp), lambda i: (0, i, 0)),
        pl.BlockSpec((R, Hp), lambda i: (i, 0)),
        pl.BlockSpec((R, 1), lambda i: (i, 0)),
        pl.BlockSpec((1, Hp), lambda i: (0, 0)),
        pl.BlockSpec((1, Hp), lambda i: (0, 0)),
        pl.BlockSpec((1, Hp), lambda i: (0, 0)),
    ]
    args = [sp, t, dinv, b, g, be]
    if with_res:
        in_specs.append(pl.BlockSpec((R, Hp), lambda i: (i, 0)))
        args.append(res)
    in_specs.append(pl.BlockSpec((Hp, Hn), lambda i: (0, 0)))
    args.append(W)

    return pl.pallas_call(
        body,
        grid=(N // R,),
        in_specs=in_specs,
        out_specs=pl.BlockSpec((R, Hn), lambda i: (i, 0)),
        out_shape=jax.ShapeDtypeStruct((N, Hn), jnp.float32),
    )(*args)


def _tc_final(sp, t, dinv, b, R=1000):
    N, Cp = t.shape
    assert N % R == 0

    def body(sp_ref, t_ref, dinv_ref, b_ref, out_ref):
        a = sp_ref[...]
        u = (a[0] + a[1] + t_ref[...]) * dinv_ref[...] + b_ref[...]
        m = jnp.max(u, axis=1, keepdims=True)
        z = u - m
        out_ref[...] = z - jnp.log(jnp.sum(jnp.exp(z), axis=1, keepdims=True))

    return pl.pallas_call(
        body,
        grid=(N // R,),
        in_specs=[
            pl.BlockSpec((NC, R, Cp), lambda i: (0, i, 0)),
            pl.BlockSpec((R, Cp), lambda i: (i, 0)),
            pl.BlockSpec((R, 1), lambda i: (i, 0)),
            pl.BlockSpec((1, Cp), lambda i: (0, 0)),
        ],
        out_specs=pl.BlockSpec((R, Cp), lambda i: (i, 0)),
        out_shape=jax.ShapeDtypeStruct((N, Cp), jnp.float32),
    )(sp, t, dinv, b)


def kernel(x, edge_index, W1, b1, g1, be1, W2, b2, g2, be2, W3, b3):
    N, F = x.shape
    H = W1.shape[1]
    C = W3.shape[1]
    E = edge_index.shape[1]
    src = edge_index[0]
    dst = edge_index[1]

    Cp = ((C + L - 1) // L) * L
    W3p = jnp.pad(W3, ((0, 0), (0, Cp - C)))
    b3p = jnp.concatenate([b3, jnp.full((Cp - C,), _NEG, jnp.float32)])

    b1r, g1r, be1r = b1[None, :], g1[None, :], be1[None, :]
    b2r, g2r, be2r = b2[None, :], g2[None, :], be2[None, :]
    b3r = b3p[None, :]

    degp = _make_deg_kernel(E, N)(dst)
    dinv, t1 = _tc_prep(degp.T, x, W1)

    K = 40
    scat_h = _make_scatter_kernel(E, N, H, K)
    scat_c = _make_scatter_kernel(E, N, Cp, K)
    src2 = src.reshape(E // K, K)
    dst2 = dst.reshape(E // K, K)

    s1 = scat_h(t1, src2, dst2)
    t2 = _tc_mid(s1, t1, dinv, b1r, g1r, be1r, x, W2)
    s2 = scat_h(t2, src2, dst2)
    t3 = _tc_mid(s2, t2, dinv, b2r, g2r, be2r, None, W3p)
    s3 = scat_c(t3, src2, dst2)
    out = _tc_final(s3, t3, dinv, b3r)
    return out[:, :C]


# speedup vs baseline: 27.9763x; 1.4752x over previous
import functools
import math

import jax
import jax.numpy as jnp
from jax import lax
from jax.experimental import pallas as pl
from jax.experimental.pallas import tpu as pltpu
from jax.experimental.pallas import tpu_sc as plsc

NC = 2
NS = 16
NW = NC * NS
L = 16

_EPS = 1e-5
_NEG = -1e30


@functools.lru_cache(maxsize=None)
def _make_deg_kernel(E, N):
    EPW = E // NW
    assert E % NW == 0 and EPW % L == 0 and N % L == 0
    mesh = plsc.VectorSubcoreMesh(core_axis_name="c", subcore_axis_name="s")

    @functools.partial(
        pl.kernel,
        out_type=jax.ShapeDtypeStruct((NW, N), jnp.float32),
        mesh=mesh,
        scratch_types=[
            pltpu.VMEM((EPW,), jnp.int32),
            pltpu.VMEM((N,), jnp.float32),
        ],
        compiler_params=pltpu.CompilerParams(needs_layout_passes=False),
    )
    def deg_kernel(dst_hbm, out_hbm, idx_v, hist_v):
        c = lax.axis_index("c")
        s = lax.axis_index("s")
        wid = s * NC + c
        zeros = jnp.zeros((L,), jnp.float32)

        def zero_body(i, carry):
            hist_v[pl.ds(i * L, L)] = zeros
            return carry

        lax.fori_loop(0, N // L, zero_body, 0)
        pltpu.sync_copy(dst_hbm.at[pl.ds(wid * EPW, EPW)], idx_v)
        ones = jnp.ones((L,), jnp.float32)

        def body(i, carry):
            idx = idx_v[pl.ds(i * L, L)]
            plsc.addupdate_scatter(hist_v, [idx], ones)
            return carry

        lax.fori_loop(0, EPW // L, body, 0)
        pltpu.sync_copy(hist_v, out_hbm.at[wid])

    return deg_kernel


@functools.lru_cache(maxsize=None)
def _make_scatter_kernel(E, N, D, K=40, ZR=25, NB=5):
    EPW = E // NW
    NCH = EPW // K
    RPT = N // NS
    assert E % NW == 0 and EPW % K == 0 and K % 8 == 0 and K <= 128
    assert N % NS == 0 and RPT % ZR == 0 and D % L == 0 and NCH % NB == 0
    assert K >= ZR
    mesh = plsc.VectorSubcoreMesh(core_axis_name="c", subcore_axis_name="s")

    @functools.partial(
        pl.kernel,
        out_type=jax.ShapeDtypeStruct((NC, N, D), jnp.float32),
        mesh=mesh,
        scratch_types=[
            pltpu.VMEM((NCH, K), jnp.int32),
            pltpu.VMEM((NCH, K), jnp.int32),
            [pltpu.VMEM((K, D), jnp.float32)] * NB,
            pltpu.VMEM_SHARED((N, D), jnp.float32),
            [pltpu.SemaphoreType.DMA] * NB,
            [pltpu.SemaphoreType.DMA] * NB,
        ],
        compiler_params=pltpu.CompilerParams(use_tc_tiling_on_sc=False),
    )
    def scatter_kernel(t_hbm, src_hbm, dst_hbm, out_hbm,
                       srcall, dstall, rows, acc_sh, gsem, ssem):
        c = lax.axis_index("c")
        s = lax.axis_index("s")
        wid = s * NC + c
        pltpu.sync_copy(src_hbm.at[pl.ds(wid * NCH, NCH)], srcall)
        pltpu.sync_copy(dst_hbm.at[pl.ds(wid * NCH, NCH)], dstall)
        zeros = jnp.zeros((L,), jnp.float32)

        def zfill_body(r, carry):
            for j in range(D // L):
                rows[0][r, pl.ds(j * L, L)] = zeros
            return carry

        lax.fori_loop(0, ZR, zfill_body, 0)

        def zero_body(i, carry):
            pltpu.sync_copy(rows[0].at[pl.ds(0, ZR)],
                            acc_sh.at[pl.ds(s * RPT + i * ZR, ZR)])
            return carry

        lax.fori_loop(0, RPT // ZR, zero_body, 0)
        plsc.subcore_barrier()

        for b in range(NB):
            pltpu.async_copy(t_hbm.at[srcall.at[b]], rows[b], gsem[b])

        NP = NCH // NB

        def body(p, carry):
            c0 = p * NB
            for b in range(NB):
                pltpu.make_async_copy(
                    t_hbm.at[srcall.at[c0 + b]], rows[b], gsem[b]).wait()
                pltpu.async_copy(
                    rows[b], acc_sh.at[dstall.at[c0 + b]], ssem[b], add=True)

            @pl.when(p < NP - 1)
            def _prefetch():
                for b in range(NB):
                    pltpu.make_async_copy(
                        rows[b], acc_sh.at[dstall.at[c0 + b]], ssem[b]).wait()
                    pltpu.async_copy(
                        t_hbm.at[srcall.at[c0 + NB + b]], rows[b], gsem[b])

            return carry

        lax.fori_loop(0, NP, body, 0)
        for b in range(NB):
            pltpu.make_async_copy(
                rows[b], acc_sh.at[dstall.at[NCH - NB + b]], ssem[b]).wait()
        plsc.subcore_barrier()
        pltpu.sync_copy(acc_sh.at[pl.ds(s * RPT, RPT)],
                        out_hbm.at[c, pl.ds(s * RPT, RPT)])

    return scatter_kernel


def _tc_prep(degp_t, x, W1, R=1000):
    N, F = x.shape
    H = W1.shape[1]
    assert N % R == 0

    def body(degp_ref, x_ref, w_ref, dinv_ref, t_ref):
        deg = jnp.sum(degp_ref[...], axis=1, keepdims=True) + 1.0
        dinv = lax.rsqrt(deg)
        dinv_ref[...] = dinv
        t_ref[...] = jnp.dot(x_ref[...], w_ref[...],
                             preferred_element_type=jnp.float32) * dinv

    return pl.pallas_call(
        body,
        grid=(N // R,),
        in_specs=[
            pl.BlockSpec((R, NW), lambda i: (i, 0)),
            pl.BlockSpec((R, F), lambda i: (i, 0)),
            pl.BlockSpec((F, H), lambda i: (0, 0)),
        ],
        out_specs=[
            pl.BlockSpec((R, 1), lambda i: (i, 0)),
            pl.BlockSpec((R, H), lambda i: (i, 0)),
        ],
        out_shape=[
            jax.ShapeDtypeStruct((N, 1), jnp.float32),
            jax.ShapeDtypeStruct((N, H), jnp.float32),
        ],
    )(degp_t, x, W1)


def _tc_mid(sp, t, dinv, b, g, be, res, W, R=1000):
    N, Hp = t.shape
    Hn = W.shape[1]
    assert N % R == 0
    bnscale = 1.0 / math.sqrt(1.0 + _EPS)
    with_res = res is not None

    def body(sp_ref, t_ref, dinv_ref, b_ref, g_ref, be_ref, *rest):
        if with_res:
            res_ref, w_ref, out_ref = rest
        else:
            w_ref, out_ref = rest
        a = sp_ref[...]
        dinv = dinv_ref[...]
        u = (a[0] + a[1] + t_ref[...]) * dinv + b_ref[...]
        u = u * (g_ref[...] * bnscale) + be_ref[...]
        if with_res:
            u = u + res_ref[...]
        u = jnp.maximum(u, 0.0)
        out_ref[...] = jnp.dot(u, w_ref[...],
                               preferred_element_type=jnp.float32) * dinv

    in_specs = [
        pl.BlockSpec((NC, R, Hp), lambda i: (0, i, 0)),
        pl.BlockSpec((R, Hp), lambda i: (i, 0)),
        pl.BlockSpec((R, 1), lambda i: (i, 0)),
        pl.BlockSpec((1, Hp), lambda i: (0, 0)),
        pl.BlockSpec((1, Hp), lambda i: (0, 0)),
        pl.BlockSpec((1, Hp), lambda i: (0, 0)),
    ]
    args = [sp, t, dinv, b, g, be]
    if with_res:
        in_specs.append(pl.BlockSpec((R, Hp), lambda i: (i, 0)))
        args.append(res)
    in_specs.append(pl.BlockSpec((Hp, Hn), lambda i: (0, 0)))
    args.append(W)

    return pl.pallas_call(
        body,
        grid=(N // R,),
        in_specs=in_specs,
        out_specs=pl.BlockSpec((R, Hn), lambda i: (i, 0)),
        out_shape=jax.ShapeDtypeStruct((N, Hn), jnp.float32),
    )(*args)


def _tc_final(sp, t, dinv, b, R=1000):
    N, Cp = t.shape
    assert N % R == 0

    def body(sp_ref, t_ref, dinv_ref, b_ref, out_ref):
        a = sp_ref[...]
        u = (a[0] + a[1] + t_ref[...]) * dinv_ref[...] + b_ref[...]
        m = jnp.max(u, axis=1, keepdims=True)
        z = u - m
        out_ref[...] = z - jnp.log(jnp.sum(jnp.exp(z), axis=1, keepdims=True))

    return pl.pallas_call(
        body,
        grid=(N // R,),
        in_specs=[
            pl.BlockSpec((NC, R, Cp), lambda i: (0, i, 0)),
            pl.BlockSpec((R, Cp), lambda i: (i, 0)),
            pl.BlockSpec((R, 1), lambda i: (i, 0)),
            pl.BlockSpec((1, Cp), lambda i: (0, 0)),
        ],
        out_specs=pl.BlockSpec((R, Cp), lambda i: (i, 0)),
        out_shape=jax.ShapeDtypeStruct((N, Cp), jnp.float32),
    )(sp, t, dinv, b)


def kernel(x, edge_index, W1, b1, g1, be1, W2, b2, g2, be2, W3, b3):
    N, F = x.shape
    H = W1.shape[1]
    C = W3.shape[1]
    E = edge_index.shape[1]
    src = edge_index[0]
    dst = edge_index[1]

    Cp = ((C + L - 1) // L) * L
    W3p = jnp.pad(W3, ((0, 0), (0, Cp - C)))
    b3p = jnp.concatenate([b3, jnp.full((Cp - C,), _NEG, jnp.float32)])

    b1r, g1r, be1r = b1[None, :], g1[None, :], be1[None, :]
    b2r, g2r, be2r = b2[None, :], g2[None, :], be2[None, :]
    b3r = b3p[None, :]

    degp = _make_deg_kernel(E, N)(dst)
    dinv, t1 = _tc_prep(degp.T, x, W1)

    K = 40
    scat_h = _make_scatter_kernel(E, N, H, K)
    scat_c = _make_scatter_kernel(E, N, Cp, K)
    src2 = src.reshape(E // K, K)
    dst2 = dst.reshape(E // K, K)

    s1 = scat_h(t1, src2, dst2)
    t2 = _tc_mid(s1, t1, dinv, b1r, g1r, be1r, x, W2)
    s2 = scat_h(t2, src2, dst2)
    t3 = _tc_mid(s2, t2, dinv, b2r, g2r, be2r, None, W3p)
    s3 = scat_c(t3, src2, dst2)
    out = _tc_final(s3, t3, dinv, b3r)
    return out[:, :C]


# speedup vs baseline: 29.5322x; 1.0556x over previous
import functools
import math

import jax
import jax.numpy as jnp
from jax import lax
from jax.experimental import pallas as pl
from jax.experimental.pallas import tpu as pltpu
from jax.experimental.pallas import tpu_sc as plsc

NC = 2
NS = 16
NW = NC * NS
L = 16

_EPS = 1e-5
_NEG = -1e30


@functools.lru_cache(maxsize=None)
def _make_deg_kernel(E, N):
    EPW = E // NW
    assert E % NW == 0 and EPW % L == 0 and N % L == 0
    mesh = plsc.VectorSubcoreMesh(core_axis_name="c", subcore_axis_name="s")

    @functools.partial(
        pl.kernel,
        out_type=jax.ShapeDtypeStruct((NW, N), jnp.float32),
        mesh=mesh,
        scratch_types=[
            pltpu.VMEM((EPW,), jnp.int32),
            pltpu.VMEM((N,), jnp.float32),
        ],
        compiler_params=pltpu.CompilerParams(needs_layout_passes=False),
    )
    def deg_kernel(dst_hbm, out_hbm, idx_v, hist_v):
        c = lax.axis_index("c")
        s = lax.axis_index("s")
        wid = s * NC + c
        zeros = jnp.zeros((L,), jnp.float32)

        def zero_body(i, carry):
            hist_v[pl.ds(i * L, L)] = zeros
            return carry

        lax.fori_loop(0, N // L, zero_body, 0)
        pltpu.sync_copy(dst_hbm.at[pl.ds(wid * EPW, EPW)], idx_v)
        ones = jnp.ones((L,), jnp.float32)

        def body(i, carry):
            idx = idx_v[pl.ds(i * L, L)]
            plsc.addupdate_scatter(hist_v, [idx], ones)
            return carry

        lax.fori_loop(0, EPW // L, body, 0)
        pltpu.sync_copy(hist_v, out_hbm.at[wid])

    return deg_kernel


@functools.lru_cache(maxsize=None)
def _make_scatter_kernel(E, N, D, K=40, ZR=25, NB=5):
    EPW = E // NW
    NCH = EPW // K
    RPT = N // NS
    assert E % NW == 0 and EPW % K == 0 and K % 8 == 0 and K <= 128
    assert N % NS == 0 and RPT % ZR == 0 and D % L == 0 and NCH % NB == 0
    assert K >= ZR
    mesh = plsc.VectorSubcoreMesh(core_axis_name="c", subcore_axis_name="s")

    @functools.partial(
        pl.kernel,
        out_type=jax.ShapeDtypeStruct((NC, N, D), jnp.float32),
        mesh=mesh,
        scratch_types=[
            pltpu.VMEM((NCH, K), jnp.int32),
            pltpu.VMEM((NCH, K), jnp.int32),
            [pltpu.VMEM((K, D), jnp.float32)] * NB,
            pltpu.VMEM_SHARED((N, D), jnp.float32),
            [pltpu.SemaphoreType.DMA] * NB,
            [pltpu.SemaphoreType.DMA] * NB,
        ],
        compiler_params=pltpu.CompilerParams(use_tc_tiling_on_sc=False),
    )
    def scatter_kernel(t_hbm, src_hbm, dst_hbm, out_hbm,
                       srcall, dstall, rows, acc_sh, gsem, ssem):
        c = lax.axis_index("c")
        s = lax.axis_index("s")
        wid = s * NC + c
        pltpu.sync_copy(src_hbm.at[pl.ds(wid * NCH, NCH)], srcall)
        pltpu.sync_copy(dst_hbm.at[pl.ds(wid * NCH, NCH)], dstall)
        zeros = jnp.zeros((L,), jnp.float32)

        def zfill_body(r, carry):
            for j in range(D // L):
                rows[0][r, pl.ds(j * L, L)] = zeros
            return carry

        lax.fori_loop(0, ZR, zfill_body, 0)

        def zero_body(i, carry):
            pltpu.sync_copy(rows[0].at[pl.ds(0, ZR)],
                            acc_sh.at[pl.ds(s * RPT + i * ZR, ZR)])
            return carry

        lax.fori_loop(0, RPT // ZR, zero_body, 0)
        plsc.subcore_barrier()

        for b in range(NB):
            pltpu.async_copy(t_hbm.at[srcall.at[b]], rows[b], gsem[b])

        NP = NCH // NB

        def body(p, carry):
            c0 = p * NB
            for b in range(NB):
                pltpu.make_async_copy(
                    t_hbm.at[srcall.at[c0 + b]], rows[b], gsem[b]).wait()
                pltpu.async_copy(
                    rows[b], acc_sh.at[dstall.at[c0 + b]], ssem[b], add=True)

            @pl.when(p < NP - 1)
            def _prefetch():
                for b in range(NB):
                    pltpu.make_async_copy(
                        rows[b], acc_sh.at[dstall.at[c0 + b]], ssem[b]).wait()
                    pltpu.async_copy(
                        t_hbm.at[srcall.at[c0 + NB + b]], rows[b], gsem[b])

            return carry

        lax.fori_loop(0, NP, body, 0)
        for b in range(NB):
            pltpu.make_async_copy(
                rows[b], acc_sh.at[dstall.at[NCH - NB + b]], ssem[b]).wait()
        plsc.subcore_barrier()
        pltpu.sync_copy(acc_sh.at[pl.ds(s * RPT, RPT)],
                        out_hbm.at[c, pl.ds(s * RPT, RPT)])

    return scatter_kernel


def _tc_prep(degp_t, x, W1, R=2000):
    N, F = x.shape
    H = W1.shape[1]
    assert N % R == 0

    def body(degp_ref, x_ref, w_ref, dinv_ref, t_ref):
        deg = jnp.sum(degp_ref[...], axis=1, keepdims=True) + 1.0
        dinv = lax.rsqrt(deg)
        dinv_ref[...] = dinv
        t_ref[...] = jnp.dot(x_ref[...], w_ref[...],
                             preferred_element_type=jnp.float32) * dinv

    return pl.pallas_call(
        body,
        grid=(N // R,),
        in_specs=[
            pl.BlockSpec((R, NW), lambda i: (i, 0)),
            pl.BlockSpec((R, F), lambda i: (i, 0)),
            pl.BlockSpec((F, H), lambda i: (0, 0)),
        ],
        out_specs=[
            pl.BlockSpec((R, 1), lambda i: (i, 0)),
            pl.BlockSpec((R, H), lambda i: (i, 0)),
        ],
        out_shape=[
            jax.ShapeDtypeStruct((N, 1), jnp.float32),
            jax.ShapeDtypeStruct((N, H), jnp.float32),
        ],
    )(degp_t, x, W1)


def _tc_mid(sp, t, dinv, b, g, be, res, W, R=2000):
    N, Hp = t.shape
    Hn = W.shape[1]
    assert N % R == 0
    bnscale = 1.0 / math.sqrt(1.0 + _EPS)
    with_res = res is not None

    def body(sp_ref, t_ref, dinv_ref, b_ref, g_ref, be_ref, *rest):
        if with_res:
            res_ref, w_ref, out_ref = rest
        else:
            w_ref, out_ref = rest
        a = sp_ref[...]
        dinv = dinv_ref[...]
        u = (a[0] + a[1] + t_ref[...]) * dinv + b_ref[...]
        u = u * (g_ref[...] * bnscale) + be_ref[...]
        if with_res:
            u = u + res_ref[...]
        u = jnp.maximum(u, 0.0)
        out_ref[...] = jnp.dot(u, w_ref[...],
                               preferred_element_type=jnp.float32) * dinv

    in_specs = [
        pl.BlockSpec((NC, R, Hp), lambda i: (0, i, 0)),
        pl.BlockSpec((R, Hp), lambda i: (i, 0)),
        pl.BlockSpec((R, 1), lambda i: (i, 0)),
        pl.BlockSpec((1, Hp), lambda i: (0, 0)),
        pl.BlockSpec((1, Hp), lambda i: (0, 0)),
        pl.BlockSpec((1, Hp), lambda i: (0, 0)),
    ]
    args = [sp, t, dinv, b, g, be]
    if with_res:
        in_specs.append(pl.BlockSpec((R, Hp), lambda i: (i, 0)))
        args.append(res)
    in_specs.append(pl.BlockSpec((Hp, Hn), lambda i: (0, 0)))
    args.append(W)

    return pl.pallas_call(
        body,
        grid=(N // R,),
        in_specs=in_specs,
        out_specs=pl.BlockSpec((R, Hn), lambda i: (i, 0)),
        out_shape=jax.ShapeDtypeStruct((N, Hn), jnp.float32),
    )(*args)


def _tc_final(sp, t, dinv, b, R=2000):
    N, Cp = t.shape
    assert N % R == 0

    def body(sp_ref, t_ref, dinv_ref, b_ref, out_ref):
        a = sp_ref[...]
        u = (a[0] + a[1] + t_ref[...]) * dinv_ref[...] + b_ref[...]
        m = jnp.max(u, axis=1, keepdims=True)
        z = u - m
        out_ref[...] = z - jnp.log(jnp.sum(jnp.exp(z), axis=1, keepdims=True))

    return pl.pallas_call(
        body,
        grid=(N // R,),
        in_specs=[
            pl.BlockSpec((NC, R, Cp), lambda i: (0, i, 0)),
            pl.BlockSpec((R, Cp), lambda i: (i, 0)),
            pl.BlockSpec((R, 1), lambda i: (i, 0)),
            pl.BlockSpec((1, Cp), lambda i: (0, 0)),
        ],
        out_specs=pl.BlockSpec((R, Cp), lambda i: (i, 0)),
        out_shape=jax.ShapeDtypeStruct((N, Cp), jnp.float32),
    )(sp, t, dinv, b)


def kernel(x, edge_index, W1, b1, g1, be1, W2, b2, g2, be2, W3, b3):
    N, F = x.shape
    H = W1.shape[1]
    C = W3.shape[1]
    E = edge_index.shape[1]
    src = edge_index[0]
    dst = edge_index[1]

    Cp = ((C + L - 1) // L) * L
    W3p = jnp.pad(W3, ((0, 0), (0, Cp - C)))
    b3p = jnp.concatenate([b3, jnp.full((Cp - C,), _NEG, jnp.float32)])

    b1r, g1r, be1r = b1[None, :], g1[None, :], be1[None, :]
    b2r, g2r, be2r = b2[None, :], g2[None, :], be2[None, :]
    b3r = b3p[None, :]

    degp = _make_deg_kernel(E, N)(dst)
    dinv, t1 = _tc_prep(degp.T, x, W1)

    KH, KC = 40, 80
    scat_h = _make_scatter_kernel(E, N, H, KH)
    scat_c = _make_scatter_kernel(E, N, Cp, KC)
    src2h, dst2h = src.reshape(E // KH, KH), dst.reshape(E // KH, KH)
    src2c, dst2c = src.reshape(E // KC, KC), dst.reshape(E // KC, KC)

    s1 = scat_h(t1, src2h, dst2h)
    t2 = _tc_mid(s1, t1, dinv, b1r, g1r, be1r, x, W2)
    s2 = scat_h(t2, src2h, dst2h)
    t3 = _tc_mid(s2, t2, dinv, b2r, g2r, be2r, None, W3p)
    s3 = scat_c(t3, src2c, dst2c)
    out = _tc_final(s3, t3, dinv, b3r)
    return out[:, :C]


# speedup vs baseline: 29.8803x; 1.0118x over previous
import functools
import math

import jax
import jax.numpy as jnp
from jax import lax
from jax.experimental import pallas as pl
from jax.experimental.pallas import tpu as pltpu
from jax.experimental.pallas import tpu_sc as plsc

NC = 2
NS = 16
NW = NC * NS
L = 16

_EPS = 1e-5
_NEG = -1e30


@functools.lru_cache(maxsize=None)
def _make_deg_kernel(E, N):
    EPW = E // NW
    assert E % NW == 0 and EPW % L == 0 and N % L == 0
    mesh = plsc.VectorSubcoreMesh(core_axis_name="c", subcore_axis_name="s")

    @functools.partial(
        pl.kernel,
        out_type=jax.ShapeDtypeStruct((NW, N), jnp.float32),
        mesh=mesh,
        scratch_types=[
            pltpu.VMEM((EPW,), jnp.int32),
            pltpu.VMEM((N,), jnp.float32),
        ],
        compiler_params=pltpu.CompilerParams(needs_layout_passes=False),
    )
    def deg_kernel(dst_hbm, out_hbm, idx_v, hist_v):
        c = lax.axis_index("c")
        s = lax.axis_index("s")
        wid = s * NC + c
        zeros = jnp.zeros((L,), jnp.float32)

        def zero_body(i, carry):
            hist_v[pl.ds(i * L, L)] = zeros
            return carry

        lax.fori_loop(0, N // L, zero_body, 0)
        pltpu.sync_copy(dst_hbm.at[pl.ds(wid * EPW, EPW)], idx_v)
        ones = jnp.ones((L,), jnp.float32)

        def body(i, carry):
            idx = idx_v[pl.ds(i * L, L)]
            plsc.addupdate_scatter(hist_v, [idx], ones)
            return carry

        lax.fori_loop(0, EPW // L, body, 0)
        pltpu.sync_copy(hist_v, out_hbm.at[wid])

    return deg_kernel


@functools.lru_cache(maxsize=None)
def _make_scatter_kernel(E, N, D, K=40, ZR=25, NB=5):
    EPW = E // NW
    NCH = EPW // K
    RPT = N // NS
    assert E % NW == 0 and EPW % K == 0 and K % 8 == 0 and K <= 128
    assert N % NS == 0 and RPT % ZR == 0 and D % L == 0 and NCH % NB == 0
    assert K >= ZR
    mesh = plsc.VectorSubcoreMesh(core_axis_name="c", subcore_axis_name="s")

    @functools.partial(
        pl.kernel,
        out_type=jax.ShapeDtypeStruct((NC, N, D), jnp.float32),
        mesh=mesh,
        scratch_types=[
            pltpu.VMEM((NCH, K), jnp.int32),
            pltpu.VMEM((NCH, K), jnp.int32),
            [pltpu.VMEM((K, D), jnp.float32)] * NB,
            pltpu.VMEM_SHARED((N, D), jnp.float32),
            [pltpu.SemaphoreType.DMA] * NB,
            [pltpu.SemaphoreType.DMA] * NB,
        ],
        compiler_params=pltpu.CompilerParams(use_tc_tiling_on_sc=False),
    )
    def scatter_kernel(t_hbm, src_hbm, dst_hbm, out_hbm,
                       srcall, dstall, rows, acc_sh, gsem, ssem):
        c = lax.axis_index("c")
        s = lax.axis_index("s")
        wid = s * NC + c
        pltpu.async_copy(src_hbm.at[pl.ds(wid * NCH, NCH)], srcall, gsem[0])
        pltpu.async_copy(dst_hbm.at[pl.ds(wid * NCH, NCH)], dstall, gsem[1])
        zeros = jnp.zeros((L,), jnp.float32)

        def zfill_body(r, carry):
            for j in range(D // L):
                rows[0][r, pl.ds(j * L, L)] = zeros
            return carry

        lax.fori_loop(0, ZR, zfill_body, 0)
        pltpu.make_async_copy(
            src_hbm.at[pl.ds(wid * NCH, NCH)], srcall, gsem[0]).wait()
        pltpu.make_async_copy(
            dst_hbm.at[pl.ds(wid * NCH, NCH)], dstall, gsem[1]).wait()

        for b in range(1, NB):
            pltpu.async_copy(t_hbm.at[srcall.at[b]], rows[b], gsem[b])

        def zero_body(i, carry):
            pltpu.sync_copy(rows[0].at[pl.ds(0, ZR)],
                            acc_sh.at[pl.ds(s * RPT + i * ZR, ZR)])
            return carry

        lax.fori_loop(0, RPT // ZR, zero_body, 0)
        plsc.subcore_barrier()
        pltpu.async_copy(t_hbm.at[srcall.at[0]], rows[0], gsem[0])

        NP = NCH // NB

        def body(p, carry):
            c0 = p * NB
            for b in range(NB):
                pltpu.make_async_copy(
                    t_hbm.at[srcall.at[c0 + b]], rows[b], gsem[b]).wait()
                pltpu.async_copy(
                    rows[b], acc_sh.at[dstall.at[c0 + b]], ssem[b], add=True)

            @pl.when(p < NP - 1)
            def _prefetch():
                for b in range(NB):
                    pltpu.make_async_copy(
                        rows[b], acc_sh.at[dstall.at[c0 + b]], ssem[b]).wait()
                    pltpu.async_copy(
                        t_hbm.at[srcall.at[c0 + NB + b]], rows[b], gsem[b])

            return carry

        lax.fori_loop(0, NP, body, 0)
        for b in range(NB):
            pltpu.make_async_copy(
                rows[b], acc_sh.at[dstall.at[NCH - NB + b]], ssem[b]).wait()
        plsc.subcore_barrier()
        pltpu.sync_copy(acc_sh.at[pl.ds(s * RPT, RPT)],
                        out_hbm.at[c, pl.ds(s * RPT, RPT)])

    return scatter_kernel


def _tc_prep(degp_t, x, W1, R=2000):
    N, F = x.shape
    H = W1.shape[1]
    assert N % R == 0

    def body(degp_ref, x_ref, w_ref, dinv_ref, t_ref):
        deg = jnp.sum(degp_ref[...], axis=1, keepdims=True) + 1.0
        dinv = lax.rsqrt(deg)
        dinv_ref[...] = dinv
        t_ref[...] = jnp.dot(x_ref[...], w_ref[...],
                             preferred_element_type=jnp.float32) * dinv

    return pl.pallas_call(
        body,
        grid=(N // R,),
        in_specs=[
            pl.BlockSpec((R, NW), lambda i: (i, 0)),
            pl.BlockSpec((R, F), lambda i: (i, 0)),
            pl.BlockSpec((F, H), lambda i: (0, 0)),
        ],
        out_specs=[
            pl.BlockSpec((R, 1), lambda i: (i, 0)),
            pl.BlockSpec((R, H), lambda i: (i, 0)),
        ],
        out_shape=[
            jax.ShapeDtypeStruct((N, 1), jnp.float32),
            jax.ShapeDtypeStruct((N, H), jnp.float32),
        ],
    )(degp_t, x, W1)


def _tc_mid(sp, t, dinv, b, g, be, res, W, R=2000):
    N, Hp = t.shape
    Hn = W.shape[1]
    assert N % R == 0
    bnscale = 1.0 / math.sqrt(1.0 + _EPS)
    with_res = res is not None

    def body(sp_ref, t_ref, dinv_ref, b_ref, g_ref, be_ref, *rest):
        if with_res:
            res_ref, w_ref, out_ref = rest
        else:
            w_ref, out_ref = rest
        a = sp_ref[...]
        dinv = dinv_ref[...]
        u = (a[0] + a[1] + t_ref[...]) * dinv + b_ref[...]
        u = u * (g_ref[...] * bnscale) + be_ref[...]
        if with_res:
            u = u + res_ref[...]
        u = jnp.maximum(u, 0.0)
        out_ref[...] = jnp.dot(u, w_ref[...],
                               preferred_element_type=jnp.float32) * dinv

    in_specs = [
        pl.BlockSpec((NC, R, Hp), lambda i: (0, i, 0)),
        pl.BlockSpec((R, Hp), lambda i: (i, 0)),
        pl.BlockSpec((R, 1), lambda i: (i, 0)),
        pl.BlockSpec((1, Hp), lambda i: (0, 0)),
        pl.BlockSpec((1, Hp), lambda i: (0, 0)),
        pl.BlockSpec((1, Hp), lambda i: (0, 0)),
    ]
    args = [sp, t, dinv, b, g, be]
    if with_res:
        in_specs.append(pl.BlockSpec((R, Hp), lambda i: (i, 0)))
        args.append(res)
    in_specs.append(pl.BlockSpec((Hp, Hn), lambda i: (0, 0)))
    args.append(W)

    return pl.pallas_call(
        body,
        grid=(N // R,),
        in_specs=in_specs,
        out_specs=pl.BlockSpec((R, Hn), lambda i: (i, 0)),
        out_shape=jax.ShapeDtypeStruct((N, Hn), jnp.float32),
    )(*args)


def _tc_final(sp, t, dinv, b, R=2000):
    N, Cp = t.shape
    assert N % R == 0

    def body(sp_ref, t_ref, dinv_ref, b_ref, out_ref):
        a = sp_ref[...]
        u = (a[0] + a[1] + t_ref[...]) * dinv_ref[...] + b_ref[...]
        m = jnp.max(u, axis=1, keepdims=True)
        z = u - m
        out_ref[...] = z - jnp.log(jnp.sum(jnp.exp(z), axis=1, keepdims=True))

    return pl.pallas_call(
        body,
        grid=(N // R,),
        in_specs=[
            pl.BlockSpec((NC, R, Cp), lambda i: (0, i, 0)),
            pl.BlockSpec((R, Cp), lambda i: (i, 0)),
            pl.BlockSpec((R, 1), lambda i: (i, 0)),
            pl.BlockSpec((1, Cp), lambda i: (0, 0)),
        ],
        out_specs=pl.BlockSpec((R, Cp), lambda i: (i, 0)),
        out_shape=jax.ShapeDtypeStruct((N, Cp), jnp.float32),
    )(sp, t, dinv, b)


def kernel(x, edge_index, W1, b1, g1, be1, W2, b2, g2, be2, W3, b3):
    N, F = x.shape
    H = W1.shape[1]
    C = W3.shape[1]
    E = edge_index.shape[1]
    src = edge_index[0]
    dst = edge_index[1]

    Cp = ((C + L - 1) // L) * L
    W3p = jnp.pad(W3, ((0, 0), (0, Cp - C)))
    b3p = jnp.concatenate([b3, jnp.full((Cp - C,), _NEG, jnp.float32)])

    b1r, g1r, be1r = b1[None, :], g1[None, :], be1[None, :]
    b2r, g2r, be2r = b2[None, :], g2[None, :], be2[None, :]
    b3r = b3p[None, :]

    degp = _make_deg_kernel(E, N)(dst)
    dinv, t1 = _tc_prep(degp.T, x, W1)

    KH, KC = 40, 80
    scat_h = _make_scatter_kernel(E, N, H, KH)
    scat_c = _make_scatter_kernel(E, N, Cp, KC)
    src2h, dst2h = src.reshape(E // KH, KH), dst.reshape(E // KH, KH)
    src2c, dst2c = src.reshape(E // KC, KC), dst.reshape(E // KC, KC)

    s1 = scat_h(t1, src2h, dst2h)
    t2 = _tc_mid(s1, t1, dinv, b1r, g1r, be1r, x, W2)
    s2 = scat_h(t2, src2h, dst2h)
    t3 = _tc_mid(s2, t2, dinv, b2r, g2r, be2r, None, W3p)
    s3 = scat_c(t3, src2c, dst2c)
    out = _tc_final(s3, t3, dinv, b3r)
    return out[:, :C]


# speedup vs baseline: 29.8857x; 1.0002x over previous
import functools
import math

import jax
import jax.numpy as jnp
from jax import lax
from jax.experimental import pallas as pl
from jax.experimental.pallas import tpu as pltpu
from jax.experimental.pallas import tpu_sc as plsc

NC = 2
NS = 16
NW = NC * NS
L = 16

_EPS = 1e-5
_NEG = -1e30


@functools.lru_cache(maxsize=None)
def _make_deg_kernel(E, N):
    EPW = E // NW
    assert E % NW == 0 and EPW % L == 0 and N % L == 0
    mesh = plsc.VectorSubcoreMesh(core_axis_name="c", subcore_axis_name="s")

    @functools.partial(
        pl.kernel,
        out_type=jax.ShapeDtypeStruct((NW, N), jnp.float32),
        mesh=mesh,
        scratch_types=[
            pltpu.VMEM((EPW,), jnp.int32),
            pltpu.VMEM((N,), jnp.float32),
        ],
        compiler_params=pltpu.CompilerParams(needs_layout_passes=False),
    )
    def deg_kernel(dst_hbm, out_hbm, idx_v, hist_v):
        c = lax.axis_index("c")
        s = lax.axis_index("s")
        wid = s * NC + c
        zeros = jnp.zeros((L,), jnp.float32)

        def zero_body(i, carry):
            hist_v[pl.ds(i * L, L)] = zeros
            return carry

        lax.fori_loop(0, N // L, zero_body, 0)
        pltpu.sync_copy(dst_hbm.at[pl.ds(wid * EPW, EPW)], idx_v)
        ones = jnp.ones((L,), jnp.float32)

        def body(i, carry):
            idx = idx_v[pl.ds(i * L, L)]
            plsc.addupdate_scatter(hist_v, [idx], ones)
            return carry

        lax.fori_loop(0, EPW // L, body, 0)
        pltpu.sync_copy(hist_v, out_hbm.at[wid])

    return deg_kernel


@functools.lru_cache(maxsize=None)
def _make_scatter_kernel(E, N, D, K=40, ZR=25, NB=5):
    EPW = E // NW
    NCH = EPW // K
    RPT = N // NS
    assert E % NW == 0 and EPW % K == 0 and K % 8 == 0 and K <= 128
    assert N % NS == 0 and RPT % ZR == 0 and D % L == 0 and NCH % NB == 0
    assert K >= ZR
    mesh = plsc.VectorSubcoreMesh(core_axis_name="c", subcore_axis_name="s")

    @functools.partial(
        pl.kernel,
        out_type=jax.ShapeDtypeStruct((NC, N, D), jnp.float32),
        mesh=mesh,
        scratch_types=[
            pltpu.VMEM((NCH, K), jnp.int32),
            pltpu.VMEM((NCH, K), jnp.int32),
            [pltpu.VMEM((K, D), jnp.float32)] * NB,
            pltpu.VMEM_SHARED((N, D), jnp.float32),
            [pltpu.SemaphoreType.DMA] * NB,
            [pltpu.SemaphoreType.DMA] * NB,
        ],
        compiler_params=pltpu.CompilerParams(use_tc_tiling_on_sc=False),
    )
    def scatter_kernel(t_hbm, src_hbm, dst_hbm, out_hbm,
                       srcall, dstall, rows, acc_sh, gsem, ssem):
        c = lax.axis_index("c")
        s = lax.axis_index("s")
        wid = s * NC + c
        pltpu.async_copy(src_hbm.at[pl.ds(wid * NCH, NCH)], srcall, gsem[0])
        pltpu.async_copy(dst_hbm.at[pl.ds(wid * NCH, NCH)], dstall, gsem[1])
        zeros = jnp.zeros((L,), jnp.float32)

        def zfill_body(r, carry):
            for j in range(D // L):
                rows[0][r, pl.ds(j * L, L)] = zeros
            return carry

        lax.fori_loop(0, ZR, zfill_body, 0)
        pltpu.make_async_copy(
            src_hbm.at[pl.ds(wid * NCH, NCH)], srcall, gsem[0]).wait()
        pltpu.make_async_copy(
            dst_hbm.at[pl.ds(wid * NCH, NCH)], dstall, gsem[1]).wait()

        for b in range(1, NB):
            pltpu.async_copy(t_hbm.at[srcall.at[b]], rows[b], gsem[b])

        @pl.when(c == 0)
        def _seed_t():
            pltpu.sync_copy(t_hbm.at[pl.ds(s * RPT, RPT)],
                            acc_sh.at[pl.ds(s * RPT, RPT)])

        def zero_body(i, carry):
            pltpu.sync_copy(rows[0].at[pl.ds(0, ZR)],
                            acc_sh.at[pl.ds(s * RPT + i * ZR, ZR)])
            return carry

        @pl.when(c != 0)
        def _seed_zero():
            lax.fori_loop(0, RPT // ZR, zero_body, 0)

        plsc.subcore_barrier()
        pltpu.async_copy(t_hbm.at[srcall.at[0]], rows[0], gsem[0])

        NP = NCH // NB

        def body(p, carry):
            c0 = p * NB
            for b in range(NB):
                pltpu.make_async_copy(
                    t_hbm.at[srcall.at[c0 + b]], rows[b], gsem[b]).wait()
                pltpu.async_copy(
                    rows[b], acc_sh.at[dstall.at[c0 + b]], ssem[b], add=True)

            @pl.when(p < NP - 1)
            def _prefetch():
                for b in range(NB):
                    pltpu.make_async_copy(
                        rows[b], acc_sh.at[dstall.at[c0 + b]], ssem[b]).wait()
                    pltpu.async_copy(
                        t_hbm.at[srcall.at[c0 + NB + b]], rows[b], gsem[b])

            return carry

        lax.fori_loop(0, NP, body, 0)
        for b in range(NB):
            pltpu.make_async_copy(
                rows[b], acc_sh.at[dstall.at[NCH - NB + b]], ssem[b]).wait()
        plsc.subcore_barrier()
        pltpu.sync_copy(acc_sh.at[pl.ds(s * RPT, RPT)],
                        out_hbm.at[c, pl.ds(s * RPT, RPT)])

    return scatter_kernel


def _tc_prep(degp_t, x, W1, R=2000):
    N, F = x.shape
    H = W1.shape[1]
    assert N % R == 0

    def body(degp_ref, x_ref, w_ref, dinv_ref, t_ref):
        deg = jnp.sum(degp_ref[...], axis=1, keepdims=True) + 1.0
        dinv = lax.rsqrt(deg)
        dinv_ref[...] = dinv
        t_ref[...] = jnp.dot(x_ref[...], w_ref[...],
                             preferred_element_type=jnp.float32) * dinv

    return pl.pallas_call(
        body,
        grid=(N // R,),
        in_specs=[
            pl.BlockSpec((R, NW), lambda i: (i, 0)),
            pl.BlockSpec((R, F), lambda i: (i, 0)),
            pl.BlockSpec((F, H), lambda i: (0, 0)),
        ],
        out_specs=[
            pl.BlockSpec((R, 1), lambda i: (i, 0)),
            pl.BlockSpec((R, H), lambda i: (i, 0)),
        ],
        out_shape=[
            jax.ShapeDtypeStruct((N, 1), jnp.float32),
            jax.ShapeDtypeStruct((N, H), jnp.float32),
        ],
    )(degp_t, x, W1)


def _tc_mid(sp, dinv, b, g, be, res, W, R=2000):
    _, N, Hp = sp.shape
    Hn = W.shape[1]
    assert N % R == 0
    bnscale = 1.0 / math.sqrt(1.0 + _EPS)
    with_res = res is not None

    def body(sp_ref, dinv_ref, b_ref, g_ref, be_ref, *rest):
        if with_res:
            res_ref, w_ref, out_ref = rest
        else:
            w_ref, out_ref = rest
        a = sp_ref[...]
        dinv = dinv_ref[...]
        u = (a[0] + a[1]) * dinv + b_ref[...]
        u = u * (g_ref[...] * bnscale) + be_ref[...]
        if with_res:
            u = u + res_ref[...]
        u = jnp.maximum(u, 0.0)
        out_ref[...] = jnp.dot(u, w_ref[...],
                               preferred_element_type=jnp.float32) * dinv

    in_specs = [
        pl.BlockSpec((NC, R, Hp), lambda i: (0, i, 0)),
        pl.BlockSpec((R, 1), lambda i: (i, 0)),
        pl.BlockSpec((1, Hp), lambda i: (0, 0)),
        pl.BlockSpec((1, Hp), lambda i: (0, 0)),
        pl.BlockSpec((1, Hp), lambda i: (0, 0)),
    ]
    args = [sp, dinv, b, g, be]
    if with_res:
        in_specs.append(pl.BlockSpec((R, Hp), lambda i: (i, 0)))
        args.append(res)
    in_specs.append(pl.BlockSpec((Hp, Hn), lambda i: (0, 0)))
    args.append(W)

    return pl.pallas_call(
        body,
        grid=(N // R,),
        in_specs=in_specs,
        out_specs=pl.BlockSpec((R, Hn), lambda i: (i, 0)),
        out_shape=jax.ShapeDtypeStruct((N, Hn), jnp.float32),
    )(*args)


def _tc_final(sp, dinv, b, R=2000):
    _, N, Cp = sp.shape
    assert N % R == 0

    def body(sp_ref, dinv_ref, b_ref, out_ref):
        a = sp_ref[...]
        u = (a[0] + a[1]) * dinv_ref[...] + b_ref[...]
        m = jnp.max(u, axis=1, keepdims=True)
        z = u - m
        out_ref[...] = z - jnp.log(jnp.sum(jnp.exp(z), axis=1, keepdims=True))

    return pl.pallas_call(
        body,
        grid=(N // R,),
        in_specs=[
            pl.BlockSpec((NC, R, Cp), lambda i: (0, i, 0)),
            pl.BlockSpec((R, 1), lambda i: (i, 0)),
            pl.BlockSpec((1, Cp), lambda i: (0, 0)),
        ],
        out_specs=pl.BlockSpec((R, Cp), lambda i: (i, 0)),
        out_shape=jax.ShapeDtypeStruct((N, Cp), jnp.float32),
    )(sp, dinv, b)


def kernel(x, edge_index, W1, b1, g1, be1, W2, b2, g2, be2, W3, b3):
    N, F = x.shape
    H = W1.shape[1]
    C = W3.shape[1]
    E = edge_index.shape[1]
    src = edge_index[0]
    dst = edge_index[1]

    Cp = ((C + L - 1) // L) * L
    W3p = jnp.pad(W3, ((0, 0), (0, Cp - C)))
    b3p = jnp.concatenate([b3, jnp.full((Cp - C,), _NEG, jnp.float32)])

    b1r, g1r, be1r = b1[None, :], g1[None, :], be1[None, :]
    b2r, g2r, be2r = b2[None, :], g2[None, :], be2[None, :]
    b3r = b3p[None, :]

    degp = _make_deg_kernel(E, N)(dst)
    dinv, t1 = _tc_prep(degp.T, x, W1)

    KH, KC = 40, 80
    scat_h = _make_scatter_kernel(E, N, H, KH)
    scat_c = _make_scatter_kernel(E, N, Cp, KC)
    src2h, dst2h = src.reshape(E // KH, KH), dst.reshape(E // KH, KH)
    src2c, dst2c = src.reshape(E // KC, KC), dst.reshape(E // KC, KC)

    s1 = scat_h(t1, src2h, dst2h)
    t2 = _tc_mid(s1, dinv, b1r, g1r, be1r, x, W2)
    s2 = scat_h(t2, src2h, dst2h)
    t3 = _tc_mid(s2, dinv, b2r, g2r, be2r, None, W3p)
    s3 = scat_c(t3, src2c, dst2c)
    out = _tc_final(s3, dinv, b3r)
    return out[:, :C]


# speedup vs baseline: 30.4248x; 1.0180x over previous
import functools
import math

import jax
import jax.numpy as jnp
from jax import lax
from jax.experimental import pallas as pl
from jax.experimental.pallas import tpu as pltpu
from jax.experimental.pallas import tpu_sc as plsc

NC = 2
NS = 16
NW = NC * NS
L = 16

_EPS = 1e-5
_NEG = -1e30


@functools.lru_cache(maxsize=None)
def _make_deg_kernel(E, N):
    EPW = E // NW
    assert E % NW == 0 and EPW % L == 0 and N % L == 0
    mesh = plsc.VectorSubcoreMesh(core_axis_name="c", subcore_axis_name="s")

    @functools.partial(
        pl.kernel,
        out_type=jax.ShapeDtypeStruct((NW, N), jnp.float32),
        mesh=mesh,
        scratch_types=[
            pltpu.VMEM((EPW,), jnp.int32),
            pltpu.VMEM((N,), jnp.float32),
        ],
        compiler_params=pltpu.CompilerParams(needs_layout_passes=False),
    )
    def deg_kernel(dst_hbm, out_hbm, idx_v, hist_v):
        c = lax.axis_index("c")
        s = lax.axis_index("s")
        wid = s * NC + c
        zeros = jnp.zeros((L,), jnp.float32)

        def zero_body(i, carry):
            hist_v[pl.ds(i * L, L)] = zeros
            return carry

        lax.fori_loop(0, N // L, zero_body, 0)
        pltpu.sync_copy(dst_hbm.at[pl.ds(wid * EPW, EPW)], idx_v)
        ones = jnp.ones((L,), jnp.float32)

        def body(i, carry):
            idx = idx_v[pl.ds(i * L, L)]
            plsc.addupdate_scatter(hist_v, [idx], ones)
            return carry

        lax.fori_loop(0, EPW // L, body, 0)
        pltpu.sync_copy(hist_v, out_hbm.at[wid])

    return deg_kernel


@functools.lru_cache(maxsize=None)
def _make_scatter_kernel(E, N, D, K=40, ZR=25, NB=5):
    EPW = E // NW
    NCH = EPW // K
    RPT = N // NS
    assert E % NW == 0 and EPW % K == 0 and K % 8 == 0 and K <= 128
    assert N % NS == 0 and RPT % ZR == 0 and D % L == 0 and NCH % NB == 0
    assert K >= ZR
    mesh = plsc.VectorSubcoreMesh(core_axis_name="c", subcore_axis_name="s")

    @functools.partial(
        pl.kernel,
        out_type=jax.ShapeDtypeStruct((NC, N, D), jnp.float32),
        mesh=mesh,
        scratch_types=[
            pltpu.VMEM((NCH, K), jnp.int32),
            pltpu.VMEM((NCH, K), jnp.int32),
            [pltpu.VMEM((K, D), jnp.float32)] * NB,
            pltpu.VMEM_SHARED((N, D), jnp.float32),
            [pltpu.SemaphoreType.DMA] * NB,
            [pltpu.SemaphoreType.DMA] * NB,
        ],
        compiler_params=pltpu.CompilerParams(use_tc_tiling_on_sc=False),
    )
    def scatter_kernel(t_hbm, src_hbm, dst_hbm, out_hbm,
                       srcall, dstall, rows, acc_sh, gsem, ssem):
        c = lax.axis_index("c")
        s = lax.axis_index("s")
        wid = s * NC + c
        pltpu.async_copy(src_hbm.at[pl.ds(wid * NCH, NCH)], srcall, gsem[0])
        pltpu.async_copy(dst_hbm.at[pl.ds(wid * NCH, NCH)], dstall, gsem[1])
        zeros = jnp.zeros((L,), jnp.float32)

        def zfill_body(r, carry):
            for j in range(D // L):
                rows[0][r, pl.ds(j * L, L)] = zeros
            return carry

        lax.fori_loop(0, ZR, zfill_body, 0)
        pltpu.make_async_copy(
            src_hbm.at[pl.ds(wid * NCH, NCH)], srcall, gsem[0]).wait()
        pltpu.make_async_copy(
            dst_hbm.at[pl.ds(wid * NCH, NCH)], dstall, gsem[1]).wait()

        for b in range(1, NB):
            pltpu.async_copy(t_hbm.at[srcall.at[b]], rows[b], gsem[b])

        @pl.when(c == 0)
        def _seed_t():
            pltpu.sync_copy(t_hbm.at[pl.ds(s * RPT, RPT)],
                            acc_sh.at[pl.ds(s * RPT, RPT)])

        def zero_body(i, carry):
            pltpu.sync_copy(rows[0].at[pl.ds(0, ZR)],
                            acc_sh.at[pl.ds(s * RPT + i * ZR, ZR)])
            return carry

        @pl.when(c != 0)
        def _seed_zero():
            lax.fori_loop(0, RPT // ZR, zero_body, 0)

        plsc.subcore_barrier()
        pltpu.async_copy(t_hbm.at[srcall.at[0]], rows[0], gsem[0])

        NP = NCH // NB

        def body(p, carry):
            c0 = p * NB
            for b in range(NB):
                pltpu.make_async_copy(
                    t_hbm.at[srcall.at[c0 + b]], rows[b], gsem[b]).wait()
                pltpu.async_copy(
                    rows[b], acc_sh.at[dstall.at[c0 + b]], ssem[b], add=True)

            @pl.when(p < NP - 1)
            def _prefetch():
                for b in range(NB):
                    pltpu.make_async_copy(
                        rows[b], acc_sh.at[dstall.at[c0 + b]], ssem[b]).wait()
                    pltpu.async_copy(
                        t_hbm.at[srcall.at[c0 + NB + b]], rows[b], gsem[b])

            return carry

        lax.fori_loop(0, NP, body, 0)
        for b in range(NB):
            pltpu.make_async_copy(
                rows[b], acc_sh.at[dstall.at[NCH - NB + b]], ssem[b]).wait()
        plsc.subcore_barrier()
        pltpu.sync_copy(acc_sh.at[pl.ds(s * RPT, RPT)],
                        out_hbm.at[c, pl.ds(s * RPT, RPT)])

    return scatter_kernel


def _tc_prep(degp_t, x, W1, R=5000):
    N, F = x.shape
    H = W1.shape[1]
    assert N % R == 0

    def body(degp_ref, x_ref, w_ref, dinv_ref, t_ref):
        deg = jnp.sum(degp_ref[...], axis=1, keepdims=True) + 1.0
        dinv = lax.rsqrt(deg)
        dinv_ref[...] = dinv
        t_ref[...] = jnp.dot(x_ref[...], w_ref[...],
                             preferred_element_type=jnp.float32) * dinv

    return pl.pallas_call(
        body,
        grid=(N // R,),
        in_specs=[
            pl.BlockSpec((R, NW), lambda i: (i, 0)),
            pl.BlockSpec((R, F), lambda i: (i, 0)),
            pl.BlockSpec((F, H), lambda i: (0, 0)),
        ],
        out_specs=[
            pl.BlockSpec((R, 1), lambda i: (i, 0)),
            pl.BlockSpec((R, H), lambda i: (i, 0)),
        ],
        out_shape=[
            jax.ShapeDtypeStruct((N, 1), jnp.float32),
            jax.ShapeDtypeStruct((N, H), jnp.float32),
        ],
    )(degp_t, x, W1)


def _tc_mid(sp, dinv, b, g, be, res, W, R=5000):
    _, N, Hp = sp.shape
    Hn = W.shape[1]
    assert N % R == 0
    bnscale = 1.0 / math.sqrt(1.0 + _EPS)
    with_res = res is not None

    def body(sp_ref, dinv_ref, b_ref, g_ref, be_ref, *rest):
        if with_res:
            res_ref, w_ref, out_ref = rest
        else:
            w_ref, out_ref = rest
        a = sp_ref[...]
        dinv = dinv_ref[...]
        u = (a[0] + a[1]) * dinv + b_ref[...]
        u = u * (g_ref[...] * bnscale) + be_ref[...]
        if with_res:
            u = u + res_ref[...]
        u = jnp.maximum(u, 0.0)
        out_ref[...] = jnp.dot(u, w_ref[...],
                               preferred_element_type=jnp.float32) * dinv

    in_specs = [
        pl.BlockSpec((NC, R, Hp), lambda i: (0, i, 0)),
        pl.BlockSpec((R, 1), lambda i: (i, 0)),
        pl.BlockSpec((1, Hp), lambda i: (0, 0)),
        pl.BlockSpec((1, Hp), lambda i: (0, 0)),
        pl.BlockSpec((1, Hp), lambda i: (0, 0)),
    ]
    args = [sp, dinv, b, g, be]
    if with_res:
        in_specs.append(pl.BlockSpec((R, Hp), lambda i: (i, 0)))
        args.append(res)
    in_specs.append(pl.BlockSpec((Hp, Hn), lambda i: (0, 0)))
    args.append(W)

    return pl.pallas_call(
        body,
        grid=(N // R,),
        in_specs=in_specs,
        out_specs=pl.BlockSpec((R, Hn), lambda i: (i, 0)),
        out_shape=jax.ShapeDtypeStruct((N, Hn), jnp.float32),
    )(*args)


def _tc_final(sp, dinv, b, R=5000):
    _, N, Cp = sp.shape
    assert N % R == 0

    def body(sp_ref, dinv_ref, b_ref, out_ref):
        a = sp_ref[...]
        u = (a[0] + a[1]) * dinv_ref[...] + b_ref[...]
        m = jnp.max(u, axis=1, keepdims=True)
        z = u - m
        out_ref[...] = z - jnp.log(jnp.sum(jnp.exp(z), axis=1, keepdims=True))

    return pl.pallas_call(
        body,
        grid=(N // R,),
        in_specs=[
            pl.BlockSpec((NC, R, Cp), lambda i: (0, i, 0)),
            pl.BlockSpec((R, 1), lambda i: (i, 0)),
            pl.BlockSpec((1, Cp), lambda i: (0, 0)),
        ],
        out_specs=pl.BlockSpec((R, Cp), lambda i: (i, 0)),
        out_shape=jax.ShapeDtypeStruct((N, Cp), jnp.float32),
    )(sp, dinv, b)


def kernel(x, edge_index, W1, b1, g1, be1, W2, b2, g2, be2, W3, b3):
    N, F = x.shape
    H = W1.shape[1]
    C = W3.shape[1]
    E = edge_index.shape[1]
    src = edge_index[0]
    dst = edge_index[1]

    Cp = ((C + L - 1) // L) * L
    W3p = jnp.pad(W3, ((0, 0), (0, Cp - C)))
    b3p = jnp.concatenate([b3, jnp.full((Cp - C,), _NEG, jnp.float32)])

    b1r, g1r, be1r = b1[None, :], g1[None, :], be1[None, :]
    b2r, g2r, be2r = b2[None, :], g2[None, :], be2[None, :]
    b3r = b3p[None, :]

    degp = _make_deg_kernel(E, N)(dst)
    dinv, t1 = _tc_prep(degp.T, x, W1)

    KH, KC = 40, 80
    scat_h = _make_scatter_kernel(E, N, H, KH)
    scat_c = _make_scatter_kernel(E, N, Cp, KC)
    src2h, dst2h = src.reshape(E // KH, KH), dst.reshape(E // KH, KH)
    src2c, dst2c = src.reshape(E // KC, KC), dst.reshape(E // KC, KC)

    s1 = scat_h(t1, src2h, dst2h)
    t2 = _tc_mid(s1, dinv, b1r, g1r, be1r, x, W2)
    s2 = scat_h(t2, src2h, dst2h)
    t3 = _tc_mid(s2, dinv, b2r, g2r, be2r, None, W3p)
    s3 = scat_c(t3, src2c, dst2c)
    out = _tc_final(s3, dinv, b3r)
    return out[:, :C]
